# baseline (speedup 1.0000x reference)
import functools

import jax
import jax.numpy as jnp
from jax import lax
from jax.experimental import pallas as pl
from jax.experimental.pallas import tpu as pltpu
from jax.experimental.pallas import tpu_sc as plsc

N = 10000
E = 320000
D = 128
DV = 64
DW = 80
NPAD = 10240
NC = 2
NS = 16
NW = NC * NS
CHUNK = 128
CPW = 81
EPAD = NW * CPW * CHUNK
NBUF = 3
ROWS_PER_SUB = NPAD // NS


def _tc1_body(fb, awp, awb, vww, vwb, v80_o, abt_o):
    f = fb[...]
    v = jnp.dot(f, vww[...], preferred_element_type=jnp.float32) + vwb[...]
    sp = jnp.maximum(v, 0.0) + jnp.log(1.0 + jnp.exp(-jnp.abs(v)))
    mv = v * jnp.tanh(sp)
    ones = jnp.ones((f.shape[0], DW - DV), jnp.float32)
    v80_o[...] = jnp.concatenate([mv, ones], axis=1)
    ab = lax.dot_general(awp[...], f, (((1,), (1,)), ((), ())),
                         preferred_element_type=jnp.float32)
    abt_o[...] = ab + awb[...]


def _tc1(featp, awp, awb, vww, vwb):
    nb = NPAD // 1024
    return pl.pallas_call(
        _tc1_body,
        grid=(nb,),
        in_specs=[
            pl.BlockSpec((1024, D), lambda i: (i, 0)),
            pl.BlockSpec((2, D), lambda i: (0, 0)),
            pl.BlockSpec((1, 1), lambda i: (0, 0)),
            pl.BlockSpec((D, DV), lambda i: (0, 0)),
            pl.BlockSpec((1, DV), lambda i: (0, 0)),
        ],
        out_specs=[
            pl.BlockSpec((1024, DW), lambda i: (i, 0)),
            pl.BlockSpec((2, 1024), lambda i: (0, i)),
        ],
        out_shape=[
            jax.ShapeDtypeStruct((NPAD, DW), jnp.float32),
            jax.ShapeDtypeStruct((2, NPAD), jnp.float32),
        ],
    )(featp, awp, awb, vww, vwb)


_PROBE_SPMEM = False


def _sc_body(v80_h, abt_h, srcc_h, dstc_h, out_h, *rest):
    if _PROBE_SPMEM:
        idx_d, msg, v_sh, acc_sh, sem = rest
    else:
        idx_d, msg, acc_sh, sem = rest
        v_sh = None
    cid = lax.axis_index("c")
    sid = lax.axis_index("s")
    wid = cid * NS + sid

    if _PROBE_SPMEM:
        pltpu.sync_copy(v80_h.at[pl.ds(sid * ROWS_PER_SUB, ROWS_PER_SUB)],
                        v_sh.at[pl.ds(sid * ROWS_PER_SUB, ROWS_PER_SUB)])
    pltpu.sync_copy(dstc_h.at[wid], idx_d)

    zero = jnp.zeros((16,), jnp.float32)

    @plsc.parallel_loop(0, CHUNK)
    def _zrow(r):
        for g in range(DW // 16):
            msg[r, pl.ds(g * 16, 16)] = zero

    for k in range(ROWS_PER_SUB // CHUNK):
        pltpu.sync_copy(msg, acc_sh.at[pl.ds(sid * ROWS_PER_SUB + k * CHUNK, CHUNK)])
    plsc.subcore_barrier()

    src_tab = v_sh if _PROBE_SPMEM else v80_h

    def chunk_body(c, carry):
        dma = pltpu.async_copy(src_tab.at[idx_d.at[c]], msg, sem)
        dma.wait()
        pltpu.sync_copy(msg, acc_sh.at[idx_d.at[c]], add=True)
        return carry

    lax.fori_loop(0, CPW, chunk_body, 0)
    plsc.subcore_barrier()
    pltpu.sync_copy(acc_sh.at[pl.ds(sid * ROWS_PER_SUB, ROWS_PER_SUB)],
                    out_h.at[cid, pl.ds(sid * ROWS_PER_SUB, ROWS_PER_SUB)])


def _sc_edges(v80, abt, srcc, dstc):
    mesh = plsc.VectorSubcoreMesh(core_axis_name="c", subcore_axis_name="s")
    f = pl.kernel(
        _sc_body,
        out_type=jax.ShapeDtypeStruct((NC, NPAD, DW), jnp.float32),
        mesh=mesh,
        compiler_params=pltpu.CompilerParams(
            needs_layout_passes=False, use_tc_tiling_on_sc=False),
        scratch_types=(
            [
                pltpu.VMEM((CPW, CHUNK), jnp.int32),
                pltpu.VMEM((CHUNK, DW), jnp.float32),
            ]
            + ([pltpu.VMEM_SHARED((NPAD, DW), jnp.float32)] if _PROBE_SPMEM else [])
            + [
                pltpu.VMEM_SHARED((NPAD, DW), jnp.float32),
                pltpu.SemaphoreType.DMA,
            ]
        ),
    )
    return f(v80, abt, srcc, dstc)


def _tc2_body(acc_b, h_o):
    x = acc_b[0] + acc_b[1]
    num = x[:, :DV]
    den = x[:, DV:DV + 1]
    den = jnp.where(den > 0.0, den, 1.0)
    h_o[...] = num / den


def _tc2(acc):
    nb = NPAD // 1024
    return pl.pallas_call(
        _tc2_body,
        grid=(nb,),
        in_specs=[pl.BlockSpec((2, 1024, DW), lambda i: (0, i, 0))],
        out_specs=pl.BlockSpec((1024, DV), lambda i: (i, 0)),
        out_shape=jax.ShapeDtypeStruct((NPAD, DV), jnp.float32),
    )(acc)


def kernel(feat, edge_index, aw_w, aw_b, vw_w, vw_b):
    featp = jnp.pad(feat, ((0, NPAD - N), (0, 0)))
    awp = aw_w.reshape(2, D)
    awb = (0.5 * aw_b).reshape(1, 1)
    vwb = vw_b.reshape(1, DV)
    v80, abt = _tc1(featp, awp, awb, vw_w, vwb)

    src = edge_index[0]
    dst = edge_index[1]
    pad = EPAD - E
    srcc = jnp.concatenate([src, jnp.zeros((pad,), jnp.int32)]).reshape(NW, CPW, CHUNK)
    dstc = jnp.concatenate([dst, jnp.full((pad,), N, jnp.int32)]).reshape(NW, CPW, CHUNK)

    acc = _sc_edges(v80, abt, srcc, dstc)
    h = _tc2(acc)
    return h[:N]


# speedup vs baseline: 1.7934x; 1.7934x over previous
import functools

import jax
import jax.numpy as jnp
from jax import lax
from jax.experimental import pallas as pl
from jax.experimental.pallas import tpu as pltpu
from jax.experimental.pallas import tpu_sc as plsc

N = 10000
E = 320000
D = 128
DV = 64
DW = 80
NPAD = 10240
NC = 2
NS = 16
NW = NC * NS
CHUNK = 128
CPW = 80
EPAD = NW * CPW * CHUNK
ROWS_PER_SUB = NPAD // NS


def _tc1_body(fb, awp, awb, vww, vwb, v80_o, abt_o):
    f = fb[...]
    v = jnp.dot(f, vww[...], preferred_element_type=jnp.float32) + vwb[...]
    sp = jnp.maximum(v, 0.0) + jnp.log(1.0 + jnp.exp(-jnp.abs(v)))
    mv = v * jnp.tanh(sp)
    ones = jnp.ones((f.shape[0], DW - DV), jnp.float32)
    v80_o[...] = jnp.concatenate([mv, ones], axis=1)
    ab = lax.dot_general(awp[...], f, (((1,), (1,)), ((), ())),
                         preferred_element_type=jnp.float32)
    abt_o[...] = ab + awb[...]


def _tc1(featp, awp, awb, vww, vwb):
    nb = NPAD // 1024
    return pl.pallas_call(
        _tc1_body,
        grid=(nb,),
        in_specs=[
            pl.BlockSpec((1024, D), lambda i: (i, 0)),
            pl.BlockSpec((2, D), lambda i: (0, 0)),
            pl.BlockSpec((1, 1), lambda i: (0, 0)),
            pl.BlockSpec((D, DV), lambda i: (0, 0)),
            pl.BlockSpec((1, DV), lambda i: (0, 0)),
        ],
        out_specs=[
            pl.BlockSpec((1024, DW), lambda i: (i, 0)),
            pl.BlockSpec((2, 1024), lambda i: (0, i)),
        ],
        out_shape=[
            jax.ShapeDtypeStruct((NPAD, DW), jnp.float32),
            jax.ShapeDtypeStruct((2, NPAD), jnp.float32),
        ],
    )(featp, awp, awb, vww, vwb)


def _sc_body(v80_h, abt_h, srcc_h, dstc_h, out_h,
             asrc_v, adst_v, idx_s, idx_d, msg0, msg1, wbuf, acc_sh,
             gsem0, gsem1):
    cid = lax.axis_index("c")
    sid = lax.axis_index("s")
    wid = cid * NS + sid
    msgs = (msg0, msg1)
    gsems = (gsem0, gsem1)

    pltpu.sync_copy(abt_h.at[0], asrc_v)
    pltpu.sync_copy(abt_h.at[1], adst_v)
    pltpu.sync_copy(srcc_h.at[wid], idx_s)
    pltpu.sync_copy(dstc_h.at[wid], idx_d)

    pltpu.async_copy(v80_h.at[idx_s.at[0]], msg0, gsem0)

    zero = jnp.zeros((16,), jnp.float32)

    @plsc.parallel_loop(0, CHUNK)
    def _zrow(r):
        for g in range(DW // 16):
            msg1[r, pl.ds(g * 16, 16)] = zero

    for k in range(ROWS_PER_SUB // CHUNK):
        pltpu.sync_copy(msg1, acc_sh.at[pl.ds(sid * ROWS_PER_SUB + k * CHUNK, CHUNK)])
    plsc.subcore_barrier()

    def outer(c2, carry):
        for b in range(2):
            c = 2 * c2 + b
            buf = msgs[b]

            pltpu.make_async_copy(v80_h.at[idx_s.at[c]], buf, gsems[b]).wait()
            cn = jnp.minimum(c + 1, CPW - 1)
            pltpu.async_copy(v80_h.at[idx_s.at[cn]], msgs[1 - b], gsems[1 - b])

            @plsc.parallel_loop(0, CHUNK, step=16)
            def _wgroup(e):
                si = idx_s[c, pl.ds(e, 16)]
                di = idx_d[c, pl.ds(e, 16)]
                x = plsc.load_gather(asrc_v, [si]) + plsc.load_gather(adst_v, [di])
                s = 1.0 / (1.0 + jnp.exp(-x))
                wbuf[pl.ds(e, 16)] = jnp.exp(s)

            @plsc.parallel_loop(0, CHUNK, unroll=4)
            def _emul(e):
                ws = plsc.load_gather(wbuf, [jnp.full((16,), e, jnp.int32)])
                for g in range(DW // 16):
                    buf[e, pl.ds(g * 16, 16)] = buf[e, pl.ds(g * 16, 16)] * ws

            pltpu.sync_copy(buf, acc_sh.at[idx_d.at[c]], add=True)
        return carry

    lax.fori_loop(0, CPW // 2, outer, 0)
    pltpu.make_async_copy(v80_h.at[idx_s.at[CPW - 1]], msg0, gsem0).wait()

    plsc.subcore_barrier()
    pltpu.sync_copy(acc_sh.at[pl.ds(sid * ROWS_PER_SUB, ROWS_PER_SUB)],
                    out_h.at[cid, pl.ds(sid * ROWS_PER_SUB, ROWS_PER_SUB)])


def _sc_edges(v80, abt, srcc, dstc):
    mesh = plsc.VectorSubcoreMesh(core_axis_name="c", subcore_axis_name="s")
    f = pl.kernel(
        _sc_body,
        out_type=jax.ShapeDtypeStruct((NC, NPAD, DW), jnp.float32),
        mesh=mesh,
        compiler_params=pltpu.CompilerParams(
            needs_layout_passes=False, use_tc_tiling_on_sc=False),
        scratch_types=[
            pltpu.VMEM((NPAD,), jnp.float32),
            pltpu.VMEM((NPAD,), jnp.float32),
            pltpu.VMEM((CPW, CHUNK), jnp.int32),
            pltpu.VMEM((CPW, CHUNK), jnp.int32),
            pltpu.VMEM((CHUNK, DW), jnp.float32),
            pltpu.VMEM((CHUNK, DW), jnp.float32),
            pltpu.VMEM((CHUNK,), jnp.float32),
            pltpu.VMEM_SHARED((NPAD, DW), jnp.float32),
            pltpu.SemaphoreType.DMA,
            pltpu.SemaphoreType.DMA,
        ],
    )
    return f(v80, abt, srcc, dstc)


def _tc2_body(acc_b, h_o):
    x = acc_b[0] + acc_b[1]
    num = x[:, :DV]
    den = x[:, DV:DV + 1]
    den = jnp.where(den > 0.0, den, 1.0)
    h_o[...] = num / den


def _tc2(acc):
    nb = NPAD // 1024
    return pl.pallas_call(
        _tc2_body,
        grid=(nb,),
        in_specs=[pl.BlockSpec((2, 1024, DW), lambda i: (0, i, 0))],
        out_specs=pl.BlockSpec((1024, DV), lambda i: (i, 0)),
        out_shape=jax.ShapeDtypeStruct((NPAD, DV), jnp.float32),
    )(acc)


def kernel(feat, edge_index, aw_w, aw_b, vw_w, vw_b):
    featp = jnp.pad(feat, ((0, NPAD - N), (0, 0)))
    awp = aw_w.reshape(2, D)
    awb = (0.5 * aw_b).reshape(1, 1)
    vwb = vw_b.reshape(1, DV)
    v80, abt = _tc1(featp, awp, awb, vw_w, vwb)

    src = edge_index[0]
    dst = edge_index[1]
    pad = EPAD - E
    srcc = jnp.concatenate([src, jnp.zeros((pad,), jnp.int32)]).reshape(NW, CPW, CHUNK)
    dstc = jnp.concatenate([dst, jnp.full((pad,), N, jnp.int32)]).reshape(NW, CPW, CHUNK)

    acc = _sc_edges(v80, abt, srcc, dstc)
    h = _tc2(acc)
    return h[:N]


# speedup vs baseline: 2.9130x; 1.6243x over previous
import functools

import jax
import jax.numpy as jnp
from jax import lax
from jax.experimental import pallas as pl
from jax.experimental.pallas import tpu as pltpu
from jax.experimental.pallas import tpu_sc as plsc

N = 10000
E = 320000
D = 128
DV = 64
DW = 80
NPAD = 10240
NC = 2
NS = 16
NW = NC * NS
CHUNK = 128
CPW = 80
EPAD = NW * CPW * CHUNK
ROWS_PER_SUB = NPAD // NS


def _tc1_body(fb, awp, awb, vww, vwb, v80_o, abt_o):
    f = fb[...]
    v = jnp.dot(f, vww[...], preferred_element_type=jnp.float32) + vwb[...]
    sp = jnp.maximum(v, 0.0) + jnp.log(1.0 + jnp.exp(-jnp.abs(v)))
    mv = v * jnp.tanh(sp)
    ones = jnp.ones((f.shape[0], DW - DV), jnp.float32)
    v80_o[...] = jnp.concatenate([mv, ones], axis=1)
    ab = lax.dot_general(awp[...], f, (((1,), (1,)), ((), ())),
                         preferred_element_type=jnp.float32)
    abt_o[...] = ab + awb[...]


def _tc1(featp, awp, awb, vww, vwb):
    nb = NPAD // 1024
    return pl.pallas_call(
        _tc1_body,
        grid=(nb,),
        in_specs=[
            pl.BlockSpec((1024, D), lambda i: (i, 0)),
            pl.BlockSpec((2, D), lambda i: (0, 0)),
            pl.BlockSpec((1, 1), lambda i: (0, 0)),
            pl.BlockSpec((D, DV), lambda i: (0, 0)),
            pl.BlockSpec((1, DV), lambda i: (0, 0)),
        ],
        out_specs=[
            pl.BlockSpec((1024, DW), lambda i: (i, 0)),
            pl.BlockSpec((2, 1024), lambda i: (0, i)),
        ],
        out_shape=[
            jax.ShapeDtypeStruct((NPAD, DW), jnp.float32),
            jax.ShapeDtypeStruct((2, NPAD), jnp.float32),
        ],
    )(featp, awp, awb, vww, vwb)


def _scw_body(abt_h, sd_h, w_h, asrc_v, adst_v, meta_v, wout_v):
    cid = lax.axis_index("c")
    sid = lax.axis_index("s")
    wid = cid * NS + sid

    pltpu.sync_copy(abt_h.at[0], asrc_v)
    pltpu.sync_copy(abt_h.at[1], adst_v)
    pltpu.sync_copy(sd_h.at[wid], meta_v)

    def chunk(c, carry):
        @plsc.parallel_loop(0, CHUNK, step=16)
        def _wgroup(e):
            si = meta_v[c, 0, pl.ds(e, 16)]
            di = meta_v[c, 1, pl.ds(e, 16)]
            x = plsc.load_gather(asrc_v, [si]) + plsc.load_gather(adst_v, [di])
            s = 1.0 / (1.0 + jnp.exp(-x))
            wout_v[c, pl.ds(e, 16)] = jnp.exp(s)
        return carry

    lax.fori_loop(0, CPW, chunk, 0)
    pltpu.sync_copy(wout_v, w_h.at[wid])


def _sc_weights(abt, sd):
    mesh = plsc.VectorSubcoreMesh(core_axis_name="c", subcore_axis_name="s")
    f = pl.kernel(
        _scw_body,
        out_type=jax.ShapeDtypeStruct((NW, CPW, CHUNK), jnp.float32),
        mesh=mesh,
        compiler_params=pltpu.CompilerParams(
            needs_layout_passes=False, use_tc_tiling_on_sc=False),
        scratch_types=[
            pltpu.VMEM((NPAD,), jnp.float32),
            pltpu.VMEM((NPAD,), jnp.float32),
            pltpu.VMEM((CPW, 2, CHUNK), jnp.int32),
            pltpu.VMEM((CPW, CHUNK), jnp.float32),
        ],
    )
    return f(abt, sd)


def _sc_body(v80_h, sd_h, w_h, out_h,
             sd0, sd1, w0, w1, msg0, msg1, v_sh, acc_sh,
             gsem0, gsem1, msem0, msem1):
    cid = lax.axis_index("c")
    sid = lax.axis_index("s")
    wid = cid * NS + sid
    sds = (sd0, sd1)
    ws = (w0, w1)
    msgs = (msg0, msg1)
    gsems = (gsem0, gsem1)
    msems = (msem0, msem1)

    pltpu.sync_copy(v80_h.at[pl.ds(sid * ROWS_PER_SUB, ROWS_PER_SUB)],
                    v_sh.at[pl.ds(sid * ROWS_PER_SUB, ROWS_PER_SUB)])
    pltpu.sync_copy(sd_h.at[wid, 0], sd0.at[0])
    pltpu.sync_copy(w_h.at[wid, 0], w0.at[0])
    pltpu.async_copy(sd_h.at[wid, 1], sd1.at[0], msem1)
    pltpu.async_copy(w_h.at[wid, 1], w1.at[0], msem1)

    zero = jnp.zeros((16,), jnp.float32)

    @plsc.parallel_loop(0, CHUNK)
    def _zrow(r):
        for g in range(DW // 16):
            msg1[r, pl.ds(g * 16, 16)] = zero

    for k in range(ROWS_PER_SUB // CHUNK):
        pltpu.sync_copy(msg1, acc_sh.at[pl.ds(sid * ROWS_PER_SUB + k * CHUNK, CHUNK)])
    plsc.subcore_barrier()

    pltpu.async_copy(v_sh.at[sd0.at[0, 0]], msg0, gsem0)

    def outer(c2, carry):
        for b in range(2):
            c = 2 * c2 + b
            o = 1 - b

            pltpu.make_async_copy(v_sh.at[sds[b].at[0, 0]], msgs[b], gsems[b]).wait()

            @pl.when(c + 1 < CPW)
            def _next_gather():
                pltpu.make_async_copy(sd_h.at[wid, 0], sds[o].at[0], msems[o]).wait()
                pltpu.make_async_copy(w_h.at[wid, 0], ws[o].at[0], msems[o]).wait()
                pltpu.async_copy(v_sh.at[sds[o].at[0, 0]], msgs[o], gsems[o])

            buf = msgs[b]
            wrow = ws[b]

            @plsc.parallel_loop(0, CHUNK, unroll=4)
            def _emul(e):
                wv = plsc.load_gather(wrow, [jnp.zeros((16,), jnp.int32),
                                             jnp.full((16,), e, jnp.int32)])
                for g in range(DW // 16):
                    buf[e, pl.ds(g * 16, 16)] = buf[e, pl.ds(g * 16, 16)] * wv

            pltpu.sync_copy(buf, acc_sh.at[sds[b].at[0, 1]], add=True)

            @pl.when(c + 2 < CPW)
            def _meta_prefetch():
                pltpu.async_copy(sd_h.at[wid, c + 2], sds[b].at[0], msems[b])
                pltpu.async_copy(w_h.at[wid, c + 2], ws[b].at[0], msems[b])
        return carry

    lax.fori_loop(0, CPW // 2, outer, 0)

    plsc.subcore_barrier()
    pltpu.sync_copy(acc_sh.at[pl.ds(sid * ROWS_PER_SUB, ROWS_PER_SUB)],
                    out_h.at[cid, pl.ds(sid * ROWS_PER_SUB, ROWS_PER_SUB)])


def _sc_edges(v80, sd, w):
    mesh = plsc.VectorSubcoreMesh(core_axis_name="c", subcore_axis_name="s")
    f = pl.kernel(
        _sc_body,
        out_type=jax.ShapeDtypeStruct((NC, NPAD, DW), jnp.float32),
        mesh=mesh,
        compiler_params=pltpu.CompilerParams(
            needs_layout_passes=False, use_tc_tiling_on_sc=False),
        scratch_types=[
            pltpu.VMEM((1, 2, CHUNK), jnp.int32),
            pltpu.VMEM((1, 2, CHUNK), jnp.int32),
            pltpu.VMEM((1, CHUNK), jnp.float32),
            pltpu.VMEM((1, CHUNK), jnp.float32),
            pltpu.VMEM((CHUNK, DW), jnp.float32),
            pltpu.VMEM((CHUNK, DW), jnp.float32),
            pltpu.VMEM_SHARED((NPAD, DW), jnp.float32),
            pltpu.VMEM_SHARED((NPAD, DW), jnp.float32),
            pltpu.SemaphoreType.DMA,
            pltpu.SemaphoreType.DMA,
            pltpu.SemaphoreType.DMA,
            pltpu.SemaphoreType.DMA,
        ],
    )
    return f(v80, sd, w)


def _tc2_body(acc_b, h_o):
    x = acc_b[0] + acc_b[1]
    num = x[:, :DV]
    den = x[:, DV:DV + 1]
    den = jnp.where(den > 0.0, den, 1.0)
    h_o[...] = num / den


def _tc2(acc):
    nb = NPAD // 1024
    return pl.pallas_call(
        _tc2_body,
        grid=(nb,),
        in_specs=[pl.BlockSpec((2, 1024, DW), lambda i: (0, i, 0))],
        out_specs=pl.BlockSpec((1024, DV), lambda i: (i, 0)),
        out_shape=jax.ShapeDtypeStruct((NPAD, DV), jnp.float32),
    )(acc)


def kernel(feat, edge_index, aw_w, aw_b, vw_w, vw_b):
    featp = jnp.pad(feat, ((0, NPAD - N), (0, 0)))
    awp = aw_w.reshape(2, D)
    awb = (0.5 * aw_b).reshape(1, 1)
    vwb = vw_b.reshape(1, DV)
    v80, abt = _tc1(featp, awp, awb, vw_w, vwb)

    src = edge_index[0]
    dst = edge_index[1]
    pad = EPAD - E
    srcc = jnp.concatenate([src, jnp.zeros((pad,), jnp.int32)]).reshape(NW, CPW, 1, CHUNK)
    dstc = jnp.concatenate([dst, jnp.full((pad,), N, jnp.int32)]).reshape(NW, CPW, 1, CHUNK)
    sd = jnp.concatenate([srcc, dstc], axis=2)

    w = _sc_weights(abt, sd)
    acc = _sc_edges(v80, sd, w)
    h = _tc2(acc)
    return h[:N]


# speedup vs baseline: 3.1401x; 1.0780x over previous
import functools

import jax
import jax.numpy as jnp
from jax import lax
from jax.experimental import pallas as pl
from jax.experimental.pallas import tpu as pltpu
from jax.experimental.pallas import tpu_sc as plsc

N = 10000
E = 320000
D = 128
DV = 64
DW = 80
NPAD = 10240
NC = 2
NS = 16
NW = NC * NS
CHUNK = 128
CPW = 80
EPAD = NW * CPW * CHUNK
ROWS_PER_SUB = NPAD // NS


def _tc1_body(fb, awp, awb, vww, vwb, v80_o, abt_o):
    f = fb[...]
    v = jnp.dot(f, vww[...], preferred_element_type=jnp.float32) + vwb[...]
    sp = jnp.maximum(v, 0.0) + jnp.log(1.0 + jnp.exp(-jnp.abs(v)))
    mv = v * jnp.tanh(sp)
    ones = jnp.ones((f.shape[0], DW - DV), jnp.float32)
    v80_o[...] = jnp.concatenate([mv, ones], axis=1)
    ab = lax.dot_general(awp[...], f, (((1,), (1,)), ((), ())),
                         preferred_element_type=jnp.float32)
    abt_o[...] = ab + awb[...]


def _tc1(featp, awp, awb, vww, vwb):
    nb = NPAD // 1024
    return pl.pallas_call(
        _tc1_body,
        grid=(nb,),
        in_specs=[
            pl.BlockSpec((1024, D), lambda i: (i, 0)),
            pl.BlockSpec((2, D), lambda i: (0, 0)),
            pl.BlockSpec((1, 1), lambda i: (0, 0)),
            pl.BlockSpec((D, DV), lambda i: (0, 0)),
            pl.BlockSpec((1, DV), lambda i: (0, 0)),
        ],
        out_specs=[
            pl.BlockSpec((1024, DW), lambda i: (i, 0)),
            pl.BlockSpec((2, 1024), lambda i: (0, i)),
        ],
        out_shape=[
            jax.ShapeDtypeStruct((NPAD, DW), jnp.float32),
            jax.ShapeDtypeStruct((2, NPAD), jnp.float32),
        ],
    )(featp, awp, awb, vww, vwb)


def _scw_body(abt_h, sd_h, w_h, asrc_v, adst_v, meta_v, wout_v):
    cid = lax.axis_index("c")
    sid = lax.axis_index("s")
    wid = cid * NS + sid

    pltpu.sync_copy(abt_h.at[0], asrc_v)
    pltpu.sync_copy(abt_h.at[1], adst_v)
    pltpu.sync_copy(sd_h.at[wid], meta_v)

    def chunk(c, carry):
        @plsc.parallel_loop(0, CHUNK, step=16)
        def _wgroup(e):
            si = meta_v[c, 0, pl.ds(e, 16)]
            di = meta_v[c, 1, pl.ds(e, 16)]
            x = plsc.load_gather(asrc_v, [si]) + plsc.load_gather(adst_v, [di])
            s = 1.0 / (1.0 + jnp.exp(-x))
            wout_v[c, pl.ds(e, 16)] = jnp.exp(s)
        return carry

    lax.fori_loop(0, CPW, chunk, 0)
    pltpu.sync_copy(wout_v, w_h.at[wid])


def _sc_weights(abt, sd):
    mesh = plsc.VectorSubcoreMesh(core_axis_name="c", subcore_axis_name="s")
    f = pl.kernel(
        _scw_body,
        out_type=jax.ShapeDtypeStruct((NW, CPW, CHUNK), jnp.float32),
        mesh=mesh,
        compiler_params=pltpu.CompilerParams(
            needs_layout_passes=False, use_tc_tiling_on_sc=False),
        scratch_types=[
            pltpu.VMEM((NPAD,), jnp.float32),
            pltpu.VMEM((NPAD,), jnp.float32),
            pltpu.VMEM((CPW, 2, CHUNK), jnp.int32),
            pltpu.VMEM((CPW, CHUNK), jnp.float32),
        ],
    )
    return f(abt, sd)


def _sc_body(v80_h, sd_h, w_h, out_h,
             sd0, sd1, w0, w1, msg0, msg1, drow0, drow1, v_sh, acc_sh,
             gsem0, gsem1, msem0, msem1, ssem0, ssem1):
    cid = lax.axis_index("c")
    sid = lax.axis_index("s")
    wid = cid * NS + sid
    sds = (sd0, sd1)
    ws = (w0, w1)
    msgs = (msg0, msg1)
    drows = (drow0, drow1)
    gsems = (gsem0, gsem1)
    msems = (msem0, msem1)
    ssems = (ssem0, ssem1)

    pltpu.sync_copy(v80_h.at[pl.ds(sid * ROWS_PER_SUB, ROWS_PER_SUB)],
                    v_sh.at[pl.ds(sid * ROWS_PER_SUB, ROWS_PER_SUB)])
    pltpu.sync_copy(sd_h.at[wid, 0], sd0.at[0])
    pltpu.sync_copy(w_h.at[wid, 0], w0.at[0])
    pltpu.async_copy(sd_h.at[wid, 1], sd1.at[0], msem1)
    pltpu.async_copy(w_h.at[wid, 1], w1.at[0], msem1)

    zero = jnp.zeros((16,), jnp.float32)

    @plsc.parallel_loop(0, CHUNK)
    def _zrow(r):
        for g in range(DW // 16):
            msg1[r, pl.ds(g * 16, 16)] = zero

    for k in range(ROWS_PER_SUB // CHUNK):
        pltpu.sync_copy(msg1, acc_sh.at[pl.ds(sid * ROWS_PER_SUB + k * CHUNK, CHUNK)])
    plsc.subcore_barrier()

    pltpu.async_copy(v_sh.at[sd0.at[0, 0]], msg0, gsem0)

    def outer(c2, carry):
        for b in range(2):
            c = 2 * c2 + b
            o = 1 - b

            pltpu.make_async_copy(v_sh.at[sds[b].at[0, 0]], msgs[b], gsems[b]).wait()

            @pl.when((c > 0) & (c + 1 < CPW))
            def _wait_prev_scatter():
                pltpu.make_async_copy(
                    msgs[o], acc_sh.at[drows[o].at[0]], ssems[o]).wait()

            @pl.when(c + 1 < CPW)
            def _next_gather():
                pltpu.make_async_copy(sd_h.at[wid, 0], sds[o].at[0], msems[o]).wait()
                pltpu.make_async_copy(w_h.at[wid, 0], ws[o].at[0], msems[o]).wait()
                pltpu.async_copy(v_sh.at[sds[o].at[0, 0]], msgs[o], gsems[o])

            buf = msgs[b]
            wrow = ws[b]

            @plsc.parallel_loop(0, CHUNK, unroll=4)
            def _emul(e):
                wv = plsc.load_gather(wrow, [jnp.zeros((16,), jnp.int32),
                                             jnp.full((16,), e, jnp.int32)])
                for g in range(DW // 16):
                    buf[e, pl.ds(g * 16, 16)] = buf[e, pl.ds(g * 16, 16)] * wv

            dr = drows[b]
            for g in range(CHUNK // 16):
                dr[0, pl.ds(g * 16, 16)] = sds[b][0, 1, pl.ds(g * 16, 16)]

            pltpu.async_copy(buf, acc_sh.at[dr.at[0]], ssems[b], add=True)

            @pl.when(c + 2 < CPW)
            def _meta_prefetch():
                pltpu.async_copy(sd_h.at[wid, c + 2], sds[b].at[0], msems[b])
                pltpu.async_copy(w_h.at[wid, c + 2], ws[b].at[0], msems[b])
        return carry

    lax.fori_loop(0, CPW // 2, outer, 0)

    pltpu.make_async_copy(msg0, acc_sh.at[drow0.at[0]], ssem0).wait()
    pltpu.make_async_copy(msg1, acc_sh.at[drow1.at[0]], ssem1).wait()

    plsc.subcore_barrier()
    pltpu.sync_copy(acc_sh.at[pl.ds(sid * ROWS_PER_SUB, ROWS_PER_SUB)],
                    out_h.at[cid, pl.ds(sid * ROWS_PER_SUB, ROWS_PER_SUB)])


def _sc_edges(v80, sd, w):
    mesh = plsc.VectorSubcoreMesh(core_axis_name="c", subcore_axis_name="s")
    f = pl.kernel(
        _sc_body,
        out_type=jax.ShapeDtypeStruct((NC, NPAD, DW), jnp.float32),
        mesh=mesh,
        compiler_params=pltpu.CompilerParams(
            needs_layout_passes=False, use_tc_tiling_on_sc=False),
        scratch_types=[
            pltpu.VMEM((1, 2, CHUNK), jnp.int32),
            pltpu.VMEM((1, 2, CHUNK), jnp.int32),
            pltpu.VMEM((1, CHUNK), jnp.float32),
            pltpu.VMEM((1, CHUNK), jnp.float32),
            pltpu.VMEM((CHUNK, DW), jnp.float32),
            pltpu.VMEM((CHUNK, DW), jnp.float32),
            pltpu.VMEM((1, CHUNK), jnp.int32),
            pltpu.VMEM((1, CHUNK), jnp.int32),
            pltpu.VMEM_SHARED((NPAD, DW), jnp.float32),
            pltpu.VMEM_SHARED((NPAD, DW), jnp.float32),
            pltpu.SemaphoreType.DMA,
            pltpu.SemaphoreType.DMA,
            pltpu.SemaphoreType.DMA,
            pltpu.SemaphoreType.DMA,
            pltpu.SemaphoreType.DMA,
            pltpu.SemaphoreType.DMA,
        ],
    )
    return f(v80, sd, w)


def _tc2_body(acc_b, h_o):
    x = acc_b[0] + acc_b[1]
    num = x[:, :DV]
    den = x[:, DV:DV + 1]
    den = jnp.where(den > 0.0, den, 1.0)
    h_o[...] = num / den


def _tc2(acc):
    nb = NPAD // 1024
    return pl.pallas_call(
        _tc2_body,
        grid=(nb,),
        in_specs=[pl.BlockSpec((2, 1024, DW), lambda i: (0, i, 0))],
        out_specs=pl.BlockSpec((1024, DV), lambda i: (i, 0)),
        out_shape=jax.ShapeDtypeStruct((NPAD, DV), jnp.float32),
    )(acc)


def kernel(feat, edge_index, aw_w, aw_b, vw_w, vw_b):
    featp = jnp.pad(feat, ((0, NPAD - N), (0, 0)))
    awp = aw_w.reshape(2, D)
    awb = (0.5 * aw_b).reshape(1, 1)
    vwb = vw_b.reshape(1, DV)
    v80, abt = _tc1(featp, awp, awb, vw_w, vwb)

    src = edge_index[0]
    dst = edge_index[1]
    pad = EPAD - E
    srcc = jnp.concatenate([src, jnp.zeros((pad,), jnp.int32)]).reshape(NW, CPW, 1, CHUNK)
    dstc = jnp.concatenate([dst, jnp.full((pad,), N, jnp.int32)]).reshape(NW, CPW, 1, CHUNK)
    sd = jnp.concatenate([srcc, dstc], axis=2)

    w = _sc_weights(abt, sd)
    acc = _sc_edges(v80, sd, w)
    h = _tc2(acc)
    return h[:N]
